# Initial kernel scaffold; baseline (speedup 1.0000x reference)
import functools

import jax
import jax.numpy as jnp
from jax import lax
from jax.experimental import pallas as pl
from jax.experimental.pallas import tpu as pltpu
from jax.experimental.pallas import tpu_sc as plsc

N_NODES = 10000
N_EDGES = 320000
D = 128

NC = 2
NS = 16
NW = NC * NS

EPT = N_EDGES // NW
KB = 125
NB = EPT // KB
ROWS_PER_TILE = N_NODES // NS

_MESH = plsc.VectorSubcoreMesh(core_axis_name="c", subcore_axis_name="s")


@functools.partial(
    pl.kernel,
    out_type=jax.ShapeDtypeStruct((NW, N_NODES), jnp.float32),
    mesh=_MESH,
    scratch_types=[
        pltpu.VMEM((EPT,), jnp.int32),
        pltpu.VMEM((N_NODES,), jnp.float32),
    ],
)
def _degree_kernel(dst_hbm, out_hbm, dst_v, degs_v):
    c = lax.axis_index("c")
    s = lax.axis_index("s")
    w = s * NC + c
    pltpu.sync_copy(dst_hbm.at[pl.ds(w * EPT, EPT)], dst_v)

    zeros = jnp.zeros((16,), jnp.float32)

    def zbody(i, _):
        degs_v[pl.ds(i * 16, 16)] = zeros
        return ()

    lax.fori_loop(0, N_NODES // 16, zbody, ())

    ones = jnp.ones((16,), jnp.float32)

    def body(i, _):
        idx = dst_v[pl.ds(i * 16, 16)]
        plsc.addupdate_scatter(degs_v, [idx], ones)
        return ()

    lax.fori_loop(0, EPT // 16, body, ())
    pltpu.sync_copy(degs_v, out_hbm.at[w])


def _norm_body(degsp_ref, feat_ref, h_ref, norm_ref):
    degs = jnp.sum(degsp_ref[...], axis=0)
    norm = jnp.where(degs > 0.0, lax.rsqrt(degs), 0.0)
    nm = norm[:, None]
    h_ref[...] = feat_ref[...] * nm
    norm_ref[...] = nm


_norm_call = pl.pallas_call(
    _norm_body,
    out_shape=(
        jax.ShapeDtypeStruct((N_NODES, D), jnp.float32),
        jax.ShapeDtypeStruct((N_NODES, 1), jnp.float32),
    ),
)


@functools.partial(
    pl.kernel,
    out_type=jax.ShapeDtypeStruct((NC, N_NODES, D), jnp.float32),
    mesh=_MESH,
    scratch_types=[
        pltpu.VMEM_SHARED((N_NODES, D), jnp.float32),
        pltpu.VMEM((EPT,), jnp.int32),
        pltpu.VMEM((NB, KB), jnp.int32),
        pltpu.VMEM((2, KB, D), jnp.float32),
        pltpu.SemaphoreType.DMA,
        pltpu.SemaphoreType.DMA,
    ],
)
def _agg_kernel(h_hbm, src_hbm, dst_hbm, zeros_hbm, out_hbm,
                acc, src_v, dst_v, rows, sem0, sem1):
    c = lax.axis_index("c")
    s = lax.axis_index("s")
    w = c * NS + s

    pltpu.sync_copy(src_hbm.at[pl.ds(w * EPT, EPT)], src_v)
    pltpu.sync_copy(dst_hbm.at[pl.ds(w * NB, NB)], dst_v)

    pltpu.sync_copy(zeros_hbm, acc.at[pl.ds(s * ROWS_PER_TILE, ROWS_PER_TILE)])
    plsc.subcore_barrier()

    def body(j, _):
        idx = src_v.at[pl.ds(j * KB, KB)]
        pltpu.async_copy(h_hbm.at[idx], rows.at[0], sem0).wait()
        pltpu.sync_copy(rows.at[0], acc.at[dst_v.at[j]], add=True)
        return ()

    lax.fori_loop(0, NB, body, ())

    plsc.subcore_barrier()
    pltpu.sync_copy(
        acc.at[pl.ds(s * ROWS_PER_TILE, ROWS_PER_TILE)],
        out_hbm.at[c].at[pl.ds(s * ROWS_PER_TILE, ROWS_PER_TILE)],
    )


def _linear_body(p_ref, norm_ref, wt_ref, b_ref, out_ref):
    hp = (p_ref[0] + p_ref[1]) * norm_ref[...]
    out_ref[...] = (
        jnp.dot(hp, wt_ref[...], precision=lax.Precision.HIGHEST) + b_ref[...]
    )


_linear_call = pl.pallas_call(
    _linear_body,
    out_shape=jax.ShapeDtypeStruct((N_NODES, D), jnp.float32),
)


def kernel(features, edge_index, W, b):
    ei = edge_index.astype(jnp.int32)
    src = ei[0]
    dst = ei[1]
    dst2d = dst.reshape(N_EDGES // KB, KB)

    degs_partial = _degree_kernel(dst)
    h, norm = _norm_call(degs_partial, features)
    zeros = jnp.zeros((ROWS_PER_TILE, D), jnp.float32)
    partials = _agg_kernel(h, src, dst2d, zeros)
    return _linear_call(partials, norm, W.T, b.reshape(1, D))


# speedup vs baseline: 8.4395x; 8.4395x over previous
import functools

import jax
import jax.numpy as jnp
from jax import lax
from jax.experimental import pallas as pl
from jax.experimental.pallas import tpu as pltpu
from jax.experimental.pallas import tpu_sc as plsc

N_NODES = 10000
N_EDGES = 320000
D = 128

NC = 2
NS = 16
NW = NC * NS

EPT = N_EDGES // NW
KB = 125
NB = EPT // KB
N_PAD = 10240
ROWS_PER_TILE = N_PAD // NS

def _degree_body(dst_hbm, out_hbm, dst_v, degs_v):
    c = lax.axis_index("c")
    s = lax.axis_index("s")
    w = s * NC + c
    pltpu.sync_copy(dst_hbm.at[pl.ds(w * EPT, EPT)], dst_v)

    zeros = jnp.zeros((16,), jnp.float32)

    def zbody(i, _):
        degs_v[pl.ds(i * 16, 16)] = zeros
        return ()

    lax.fori_loop(0, N_NODES // 16, zbody, ())

    ones = jnp.ones((16,), jnp.float32)

    def body(i, _):
        idx = dst_v[pl.ds(i * 16, 16)]
        plsc.addupdate_scatter(degs_v, [idx], ones)
        return ()

    lax.fori_loop(0, EPT // 16, body, ())
    pltpu.sync_copy(degs_v, out_hbm.at[pl.ds(w * N_NODES, N_NODES)])


def _norm_body(degsp_ref, feat_ref, h_ref, norm_ref):
    degs = jnp.sum(degsp_ref[...], axis=0)
    norm = jnp.where(degs > 0.0, lax.rsqrt(degs), 0.0)
    nm = norm[:, None]
    h_ref[...] = feat_ref[...] * nm
    norm_ref[...] = nm


_norm_call = pl.pallas_call(
    _norm_body,
    out_shape=(
        jax.ShapeDtypeStruct((N_NODES, D), jnp.float32),
        jax.ShapeDtypeStruct((N_NODES, 1), jnp.float32),
    ),
)


def _agg_body(h_hbm, src_hbm, dst_hbm, zeros_hbm, out_hbm,
              acc, src_v, dst_v, rows, sem0, sem1):
    c = lax.axis_index("c")
    s = lax.axis_index("s")
    w = c * NS + s

    pltpu.sync_copy(src_hbm.at[pl.ds(w * NB, NB)], src_v)
    pltpu.sync_copy(dst_hbm.at[pl.ds(w * NB, NB)], dst_v)

    pltpu.sync_copy(zeros_hbm, acc.at[pl.ds(s * ROWS_PER_TILE, ROWS_PER_TILE)])
    plsc.subcore_barrier()

    def body(j, _):
        pltpu.async_copy(h_hbm.at[src_v.at[j]], rows.at[0], sem0).wait()
        pltpu.sync_copy(rows.at[0], acc.at[dst_v.at[j]], add=True)
        return ()

    lax.fori_loop(0, NB, body, ())

    plsc.subcore_barrier()
    pltpu.sync_copy(
        acc.at[pl.ds(s * ROWS_PER_TILE, ROWS_PER_TILE)],
        out_hbm.at[c].at[pl.ds(s * ROWS_PER_TILE, ROWS_PER_TILE)],
    )


def _linear_body(p_ref, norm_ref, wt_ref, b_ref, out_ref):
    hp = (p_ref[0][:N_NODES] + p_ref[1][:N_NODES]) * norm_ref[...]
    out_ref[...] = (
        jnp.dot(hp, wt_ref[...], precision=lax.Precision.HIGHEST) + b_ref[...]
    )


_linear_call = pl.pallas_call(
    _linear_body,
    out_shape=jax.ShapeDtypeStruct((N_NODES, D), jnp.float32),
)


@functools.cache
def _sc_kernels():
    mesh = plsc.VectorSubcoreMesh(
        core_axis_name="c", subcore_axis_name="s", num_cores=NC, num_subcores=NS
    )
    params = pltpu.CompilerParams(needs_layout_passes=False)
    degree_kernel = pl.kernel(
        _degree_body,
        out_type=jax.ShapeDtypeStruct((NW * N_NODES,), jnp.float32),
        mesh=mesh,
        compiler_params=params,
        scratch_types=[
            pltpu.VMEM((EPT,), jnp.int32),
            pltpu.VMEM((N_NODES,), jnp.float32),
        ],
    )
    agg_kernel = pl.kernel(
        _agg_body,
        out_type=jax.ShapeDtypeStruct((NC, N_PAD, D), jnp.float32),
        mesh=mesh,
        compiler_params=params,
        scratch_types=[
            pltpu.VMEM_SHARED((N_PAD, D), jnp.float32),
            pltpu.VMEM((NB, KB), jnp.int32),
            pltpu.VMEM((NB, KB), jnp.int32),
            pltpu.VMEM((1, KB, D), jnp.float32),
            pltpu.SemaphoreType.DMA,
            pltpu.SemaphoreType.DMA,
        ],
    )
    return degree_kernel, agg_kernel


def kernel(features, edge_index, W, b):
    ei = edge_index.astype(jnp.int32)
    src = ei[0]
    dst = ei[1]
    src2d = src.reshape(N_EDGES // KB, KB)
    dst2d = dst.reshape(N_EDGES // KB, KB)

    degree_kernel, agg_kernel = _sc_kernels()
    degs_partial = degree_kernel(dst).reshape(NW, N_NODES)
    h, norm = _norm_call(degs_partial, features)
    zeros = jnp.zeros((ROWS_PER_TILE, D), jnp.float32)
    partials = agg_kernel(h, src2d, dst2d, zeros)
    return _linear_call(partials, norm, W.T, b.reshape(1, D))


# speedup vs baseline: 11.5901x; 1.3733x over previous
import functools

import jax
import jax.numpy as jnp
from jax import lax
from jax.experimental import pallas as pl
from jax.experimental.pallas import tpu as pltpu
from jax.experimental.pallas import tpu_sc as plsc

N_NODES = 10000
N_EDGES = 320000
D = 128

NC = 2
NS = 16
NW = NC * NS

EPT = N_EDGES // NW
KB = 125
NB = EPT // KB
N_PAD = 10240
ROWS_PER_TILE = N_PAD // NS

def _degree_body(dst_hbm, out_hbm, dst_v, degs_v):
    c = lax.axis_index("c")
    s = lax.axis_index("s")
    w = s * NC + c
    pltpu.sync_copy(dst_hbm.at[pl.ds(w * EPT, EPT)], dst_v)

    zeros = jnp.zeros((16,), jnp.float32)

    def zbody(i, _):
        degs_v[pl.ds(i * 16, 16)] = zeros
        return ()

    lax.fori_loop(0, N_NODES // 16, zbody, ())

    ones = jnp.ones((16,), jnp.float32)

    def body(i, _):
        idx = dst_v[pl.ds(i * 16, 16)]
        plsc.addupdate_scatter(degs_v, [idx], ones)
        return ()

    lax.fori_loop(0, EPT // 16, body, ())
    pltpu.sync_copy(degs_v, out_hbm.at[pl.ds(w * N_NODES, N_NODES)])


def _norm_body(degsp_ref, feat_ref, h_ref, norm_ref):
    degs = jnp.sum(degsp_ref[...], axis=0)
    norm = jnp.where(degs > 0.0, lax.rsqrt(degs), 0.0)
    nm = norm[:, None]
    h_ref[...] = feat_ref[...] * nm
    norm_ref[...] = nm


_norm_call = pl.pallas_call(
    _norm_body,
    out_shape=(
        jax.ShapeDtypeStruct((N_NODES, D), jnp.float32),
        jax.ShapeDtypeStruct((N_NODES, 1), jnp.float32),
    ),
)


HALF = NB // 2


def _agg_body(h_hbm, src_hbm, dst_hbm, zeros_hbm, out_hbm,
              acc, src_v, dst_v, rows, sem0, sem1):
    c = lax.axis_index("c")
    s = lax.axis_index("s")
    w = c * NS + s

    pltpu.sync_copy(zeros_hbm, acc.at[pl.ds(s * ROWS_PER_TILE, ROWS_PER_TILE)])
    plsc.subcore_barrier()

    def gather(j, buf, sem):
        return pltpu.async_copy(h_hbm.at[src_v.at[j]], rows.at[buf], sem)

    def run_half(hf):
        base = w * NB + hf * HALF
        pltpu.sync_copy(src_hbm.at[pl.ds(base, HALF)], src_v)
        pltpu.sync_copy(dst_hbm.at[pl.ds(base, HALF)], dst_v)
        gather(0, 0, sem0)

        def body(i, _):
            j0 = 2 * i
            j1 = j0 + 1
            gather(j1, 1, sem1)
            pltpu.make_async_copy(h_hbm.at[src_v.at[j0]], rows.at[0], sem0).wait()
            pltpu.sync_copy(rows.at[0], acc.at[dst_v.at[j0]], add=True)

            @pl.when(j0 + 2 < HALF)
            def _():
                gather(j0 + 2, 0, sem0)

            pltpu.make_async_copy(h_hbm.at[src_v.at[j1]], rows.at[1], sem1).wait()
            pltpu.sync_copy(rows.at[1], acc.at[dst_v.at[j1]], add=True)
            return ()

        lax.fori_loop(0, HALF // 2, body, ())

    run_half(0)
    run_half(1)

    plsc.subcore_barrier()
    pltpu.sync_copy(
        acc.at[pl.ds(s * ROWS_PER_TILE, ROWS_PER_TILE)],
        out_hbm.at[c].at[pl.ds(s * ROWS_PER_TILE, ROWS_PER_TILE)],
    )


def _linear_body(p_ref, norm_ref, wt_ref, b_ref, out_ref):
    hp = (p_ref[0][:N_NODES] + p_ref[1][:N_NODES]) * norm_ref[...]
    out_ref[...] = (
        jnp.dot(hp, wt_ref[...], precision=lax.Precision.HIGHEST) + b_ref[...]
    )


_linear_call = pl.pallas_call(
    _linear_body,
    out_shape=jax.ShapeDtypeStruct((N_NODES, D), jnp.float32),
)


@functools.cache
def _sc_kernels():
    mesh = plsc.VectorSubcoreMesh(
        core_axis_name="c", subcore_axis_name="s", num_cores=NC, num_subcores=NS
    )
    params = pltpu.CompilerParams(needs_layout_passes=False)
    degree_kernel = pl.kernel(
        _degree_body,
        out_type=jax.ShapeDtypeStruct((NW * N_NODES,), jnp.float32),
        mesh=mesh,
        compiler_params=params,
        scratch_types=[
            pltpu.VMEM((EPT,), jnp.int32),
            pltpu.VMEM((N_NODES,), jnp.float32),
        ],
    )
    agg_kernel = pl.kernel(
        _agg_body,
        out_type=jax.ShapeDtypeStruct((NC, N_PAD, D), jnp.float32),
        mesh=mesh,
        compiler_params=params,
        scratch_types=[
            pltpu.VMEM_SHARED((N_PAD, D), jnp.float32),
            pltpu.VMEM((HALF, KB), jnp.int32),
            pltpu.VMEM((HALF, KB), jnp.int32),
            pltpu.VMEM((2, KB, D), jnp.float32),
            pltpu.SemaphoreType.DMA,
            pltpu.SemaphoreType.DMA,
        ],
    )
    return degree_kernel, agg_kernel


def kernel(features, edge_index, W, b):
    ei = edge_index.astype(jnp.int32)
    src = ei[0]
    dst = ei[1]
    src2d = src.reshape(N_EDGES // KB, KB)
    dst2d = dst.reshape(N_EDGES // KB, KB)

    degree_kernel, agg_kernel = _sc_kernels()
    degs_partial = degree_kernel(dst).reshape(NW, N_NODES)
    h, norm = _norm_call(degs_partial, features)
    zeros = jnp.zeros((ROWS_PER_TILE, D), jnp.float32)
    partials = agg_kernel(h, src2d, dst2d, zeros)
    return _linear_call(partials, norm, W.T, b.reshape(1, D))
